# Initial kernel scaffold; baseline (speedup 1.0000x reference)
import jax
import jax.numpy as jnp
from jax.experimental import pallas as pl


def kernel(x, edge_index1, pseudo1, cluster1, edge_index2, pseudo2, cluster2, edge_index3, pseudo3, cluster3, edge_index4, pseudo4, W1, root1, b1, W2, root2, b2, W3, root3, b3, W42, root42, b42, Wfc1, bfc1, W5, root5, b5, W6, root6, b6, W7, root7, b7, Wsk1, bsk1, Wsk2, bsk2, Wsk3, bsk3, Wfc2, bfc2):
    raise NotImplementedError("write your pallas kernel here")


# speedup vs baseline: 1.1041x; 1.1041x over previous
import functools

import jax
import jax.numpy as jnp
from jax import lax
from jax.experimental import pallas as pl
from jax.experimental.pallas import tpu as pltpu

_K = 5
_PREC = lax.Precision.HIGHEST


def _ceil_to(a, m):
    return (a + m - 1) // m * m


def _conv_body(srct_ref, dst_ref, p0_ref, p1_ref, p2_ref, x_ref, wf_ref,
               root_ref, bias_ref, out_ref, acc_ref, *, nb, cw, n, cin,
               cout, elu):
    acc_ref[...] = jnp.zeros((n, cout + 1), jnp.float32)
    nt = min(n, 2500)
    tiles = [(lo, min(nt, n - lo)) for lo in range(0, n, nt)]
    iota5 = lax.broadcasted_iota(jnp.int32, (_K, 1), 0)

    def body(i, carry):
        src_cv = srct_ref[pl.ds(i, 1), :].reshape(cw, 1)
        dst_c = dst_ref[pl.ds(i, 1), :]
        x_src = jnp.zeros((cw, cin), jnp.float32)
        for lo, sz in tiles:
            it = lax.broadcasted_iota(jnp.int32, (1, sz), 1) + lo
            s_t = (src_cv == it).astype(jnp.bfloat16)
            x_src = x_src + jnp.dot(
                s_t, x_ref[pl.ds(lo, sz), :].astype(jnp.bfloat16),
                preferred_element_type=jnp.float32)
        bs = []
        for p_ref in (p0_ref, p1_ref, p2_ref):
            p = p_ref[pl.ds(i, 1), :] * (_K - 1.0)
            i0f = jnp.clip(jnp.floor(p), 0.0, _K - 2.0)
            fr = p - i0f
            i0 = i0f.astype(jnp.int32)
            b = (jnp.where(iota5 == i0, 1.0 - fr, 0.0)
                 + jnp.where(iota5 == i0 + 1, fr, 0.0))
            bs.append(b)
        m01 = (bs[0][:, None, :] * bs[1][None, :, :]).reshape(_K * _K, cw)
        cmat = (m01[:, None, :] * bs[2][None, :, :]).reshape(_K ** 3, cw)
        mixed = lax.dot_general(cmat.astype(jnp.bfloat16), wf_ref[...],
                                (((0,), (0,)), ((), ())),
                                preferred_element_type=jnp.float32)
        m = jnp.sum(mixed.reshape(cw, cin, cout) * x_src[:, :, None], axis=1)
        m_aug = jnp.concatenate(
            [m, jnp.ones((cw, 1), jnp.float32)], axis=1).astype(jnp.bfloat16)
        for lo, sz in tiles:
            it = lax.broadcasted_iota(jnp.int32, (sz, 1), 0) + lo
            p_t = (it == dst_c).astype(jnp.bfloat16)
            acc_ref[pl.ds(lo, sz), :] += jnp.dot(
                p_t, m_aug, preferred_element_type=jnp.float32)
        return carry

    lax.fori_loop(0, nb, body, 0)
    for lo, sz in tiles:
        acc = acc_ref[pl.ds(lo, sz), :cout]
        deg = acc_ref[pl.ds(lo, sz), cout:cout + 1]
        res = (acc / jnp.maximum(deg, 1.0)
               + jnp.dot(x_ref[pl.ds(lo, sz), :], root_ref[...],
                         precision=_PREC)
               + bias_ref[...])
        if elu:
            res = jnp.where(res > 0.0, res,
                            jnp.exp(jnp.minimum(res, 0.0)) - 1.0)
        out_ref[pl.ds(lo, sz), :] = res


def _spline_conv(x, ei, pseudo, w, root, bias, elu, cw=None):
    n, cin = x.shape
    if cw is None:
        cw = 128
    k3, _, cout = w.shape
    e = ei.shape[1]
    ep = _ceil_to(e, cw)
    nb = ep // cw
    pad = ep - e
    src = jnp.pad(ei[0], (0, pad)).reshape(nb, cw)
    dst = jnp.pad(ei[1], (0, pad), constant_values=-1).reshape(nb, cw)
    ps = [jnp.pad(pseudo[:, d], (0, pad)).reshape(nb, cw) for d in range(3)]
    body = functools.partial(_conv_body, nb=nb, cw=cw, n=n, cin=cin,
                             cout=cout, elu=elu)
    return pl.pallas_call(
        body,
        out_shape=jax.ShapeDtypeStruct((n, cout), jnp.float32),
        scratch_shapes=[pltpu.VMEM((n, cout + 1), jnp.float32)],
    )(src, dst, ps[0], ps[1], ps[2], x,
      w.reshape(k3, cin * cout).astype(jnp.bfloat16), root,
      bias.reshape(1, cout))


def _pool_body(cl_ref, x_ref, out_ref, *, nb, cw, nc, ch):
    out_ref[...] = jnp.full((nc, ch), -jnp.inf, jnp.float32)
    iota_nc = lax.broadcasted_iota(jnp.int32, (nc, 1), 0)

    def body(i, carry):
        cl = cl_ref[pl.ds(i, 1), :]
        xc = x_ref[pl.ds(i * cw, cw), :]
        m = (iota_nc == cl)
        best = jnp.full((nc, ch), -jnp.inf, jnp.float32)
        for r in range(cw):
            cand = jnp.where(m[:, r:r + 1], xc[r:r + 1, :], -jnp.inf)
            best = jnp.maximum(best, cand)
        out_ref[...] = jnp.maximum(out_ref[...], best)
        return carry

    lax.fori_loop(0, nb, body, 0)
    o = out_ref[...]
    out_ref[...] = jnp.where(o > -jnp.inf, o, 0.0)


def _voxel_max_pool(x, cluster, nc, cw=8):
    nf, ch = x.shape
    nfp = _ceil_to(nf, cw)
    nb = nfp // cw
    cl = jnp.pad(cluster, (0, nfp - nf), constant_values=-1).reshape(nb, cw)
    xp = jnp.pad(x, ((0, nfp - nf), (0, 0)))
    body = functools.partial(_pool_body, nb=nb, cw=cw, nc=nc, ch=ch)
    return pl.pallas_call(
        body,
        out_shape=jax.ShapeDtypeStruct((nc, ch), jnp.float32),
    )(cl, xp)


def _unpool_body(clt_ref, xc_ref, xf_ref, wsk_ref, bsk_ref, out_ref, *,
                 nb, cw, nc, chc, chs):
    iota_nc = lax.broadcasted_iota(jnp.int32, (1, nc), 1)

    def body(i, carry):
        cl_cv = clt_ref[pl.ds(i, 1), :].reshape(cw, 1)
        g = (cl_cv == iota_nc).astype(jnp.float32)
        gath = jnp.dot(g, xc_ref[...], precision=_PREC)
        skip = (jnp.dot(xf_ref[pl.ds(i * cw, cw), :], wsk_ref[...],
                        precision=_PREC) + bsk_ref[...])
        out_ref[pl.ds(i * cw, cw), :chc] = gath
        out_ref[pl.ds(i * cw, cw), chc:chc + chs] = skip
        return carry

    lax.fori_loop(0, nb, body, 0)


def _unpool_concat(xc, cluster, xf, wsk, bsk, cw=500):
    nf, chf = xf.shape
    nc, chc = xc.shape
    chs = wsk.shape[1]
    nfp = _ceil_to(nf, cw)
    nb = nfp // cw
    cl = jnp.pad(cluster, (0, nfp - nf), constant_values=-1).reshape(nb, cw)
    xfp = jnp.pad(xf, ((0, nfp - nf), (0, 0)))
    body = functools.partial(_unpool_body, nb=nb, cw=cw, nc=nc, chc=chc,
                             chs=chs)
    out = pl.pallas_call(
        body,
        out_shape=jax.ShapeDtypeStruct((nfp, chc + chs), jnp.float32),
    )(cl, xc, xfp, wsk, bsk.reshape(1, chs))
    return out[:nf]


def _dense_elu_body(x_ref, w_ref, b_ref, out_ref):
    z = jnp.dot(x_ref[...], w_ref[...], precision=_PREC) + b_ref[...]
    out_ref[...] = jnp.where(z > 0.0, z, jnp.exp(jnp.minimum(z, 0.0)) - 1.0)


def _dense_elu(x, w, b):
    return pl.pallas_call(
        _dense_elu_body,
        out_shape=jax.ShapeDtypeStruct((x.shape[0], w.shape[1]),
                                       jnp.float32),
    )(x, w, b.reshape(1, -1))


def _logsoftmax_body(x_ref, w_ref, b_ref, out_ref):
    z = jnp.dot(x_ref[...], w_ref[...], precision=_PREC) + b_ref[...]
    m = jnp.max(z, axis=1, keepdims=True)
    s = z - m
    out_ref[...] = s - jnp.log(jnp.sum(jnp.exp(s), axis=1, keepdims=True))


def _dense_logsoftmax(x, w, b):
    return pl.pallas_call(
        _logsoftmax_body,
        out_shape=jax.ShapeDtypeStruct((x.shape[0], w.shape[1]),
                                       jnp.float32),
    )(x, w, b.reshape(1, -1))


def kernel(x, edge_index1, pseudo1, cluster1, edge_index2, pseudo2,
           cluster2, edge_index3, pseudo3, cluster3, edge_index4, pseudo4,
           W1, root1, b1, W2, root2, b2, W3, root3, b3, W42, root42, b42,
           Wfc1, bfc1, W5, root5, b5, W6, root6, b6, W7, root7, b7,
           Wsk1, bsk1, Wsk2, bsk2, Wsk3, bsk3, Wfc2, bfc2):
    x1 = _spline_conv(x, edge_index1, pseudo1, W1, root1, b1, True)
    x2 = _voxel_max_pool(x1, cluster1, 2500)
    x2 = _spline_conv(x2, edge_index2, pseudo2, W2, root2, b2, True)
    x3 = _voxel_max_pool(x2, cluster2, 600)
    x3 = _spline_conv(x3, edge_index3, pseudo3, W3, root3, b3, True)
    x4 = _voxel_max_pool(x3, cluster3, 150)
    x4 = _spline_conv(x4, edge_index4, pseudo4, W42, root42, b42, True)
    x4 = _dense_elu(x4, Wfc1, bfc1)
    x3c = _unpool_concat(x4, cluster3, x3, Wsk3, bsk3)
    x3c = _spline_conv(x3c, edge_index3, pseudo3, W5, root5, b5, True)
    x2c = _unpool_concat(x3c, cluster2, x2, Wsk2, bsk2)
    x2c = _spline_conv(x2c, edge_index2, pseudo2, W6, root6, b6, True)
    x1c = _unpool_concat(x2c, cluster1, x1, Wsk1, bsk1)
    x1c = _spline_conv(x1c, edge_index1, pseudo1, W7, root7, b7, True)
    return _dense_logsoftmax(x1c, Wfc2, bfc2)


# speedup vs baseline: 1.3932x; 1.2618x over previous
import functools

import jax
import jax.numpy as jnp
from jax import lax
from jax.experimental import pallas as pl
from jax.experimental.pallas import tpu as pltpu

_K = 5
_PREC = lax.Precision.HIGHEST


def _ceil_to(a, m):
    return (a + m - 1) // m * m


def _conv_body(srct_ref, dst_ref, p0_ref, p1_ref, p2_ref, x_ref, wf_ref,
               root_ref, bias_ref, out_ref, acc_ref, *, nb, cw, n, cin,
               cout, elu):
    acc_ref[...] = jnp.zeros((n, cout + 1), jnp.float32)
    nt = min(n, 2500)
    tiles = [(lo, min(nt, n - lo)) for lo in range(0, n, nt)]
    iota5 = lax.broadcasted_iota(jnp.int32, (_K, 1), 0)

    def body(i, carry):
        src_cv = srct_ref[pl.ds(i, 1), :].reshape(cw, 1)
        dst_c = dst_ref[pl.ds(i, 1), :]
        x_src = jnp.zeros((cw, cin), jnp.float32)
        for lo, sz in tiles:
            it = lax.broadcasted_iota(jnp.int32, (1, sz), 1) + lo
            s_t = (src_cv == it).astype(jnp.bfloat16)
            x_src = x_src + jnp.dot(
                s_t, x_ref[pl.ds(lo, sz), :].astype(jnp.bfloat16),
                preferred_element_type=jnp.float32)
        bs = []
        for p_ref in (p0_ref, p1_ref, p2_ref):
            p = p_ref[pl.ds(i, 1), :] * (_K - 1.0)
            i0f = jnp.clip(jnp.floor(p), 0.0, _K - 2.0)
            fr = p - i0f
            i0 = i0f.astype(jnp.int32)
            b = (jnp.where(iota5 == i0, 1.0 - fr, 0.0)
                 + jnp.where(iota5 == i0 + 1, fr, 0.0))
            bs.append(b)
        m01 = (bs[0][:, None, :] * bs[1][None, :, :]).reshape(_K * _K, cw)
        cmat = (m01[:, None, :] * bs[2][None, :, :]).reshape(_K ** 3, cw)
        mixed = lax.dot_general(cmat.astype(jnp.bfloat16), wf_ref[...],
                                (((0,), (0,)), ((), ())),
                                preferred_element_type=jnp.float32)
        m = jnp.sum(mixed.reshape(cw, cin, cout) * x_src[:, :, None], axis=1)
        m_aug = jnp.concatenate(
            [m, jnp.ones((cw, 1), jnp.float32)], axis=1).astype(jnp.bfloat16)
        for lo, sz in tiles:
            it = lax.broadcasted_iota(jnp.int32, (sz, 1), 0) + lo
            p_t = (it == dst_c).astype(jnp.bfloat16)
            acc_ref[pl.ds(lo, sz), :] += jnp.dot(
                p_t, m_aug, preferred_element_type=jnp.float32)
        return carry

    lax.fori_loop(0, nb, body, 0)
    for lo, sz in tiles:
        acc = acc_ref[pl.ds(lo, sz), :cout]
        deg = acc_ref[pl.ds(lo, sz), cout:cout + 1]
        res = (acc / jnp.maximum(deg, 1.0)
               + jnp.dot(x_ref[pl.ds(lo, sz), :], root_ref[...],
                         precision=_PREC)
               + bias_ref[...])
        if elu:
            res = jnp.where(res > 0.0, res,
                            jnp.exp(jnp.minimum(res, 0.0)) - 1.0)
        out_ref[pl.ds(lo, sz), :] = res


def _spline_conv(x, ei, pseudo, w, root, bias, elu, cw=None):
    n, cin = x.shape
    if cw is None:
        cw = 256
    k3, _, cout = w.shape
    e = ei.shape[1]
    ep = _ceil_to(e, cw)
    nb = ep // cw
    pad = ep - e
    src = jnp.pad(ei[0], (0, pad)).reshape(nb, cw)
    dst = jnp.pad(ei[1], (0, pad), constant_values=-1).reshape(nb, cw)
    ps = [jnp.pad(pseudo[:, d], (0, pad)).reshape(nb, cw) for d in range(3)]
    body = functools.partial(_conv_body, nb=nb, cw=cw, n=n, cin=cin,
                             cout=cout, elu=elu)
    return pl.pallas_call(
        body,
        out_shape=jax.ShapeDtypeStruct((n, cout), jnp.float32),
        scratch_shapes=[pltpu.VMEM((n, cout + 1), jnp.float32)],
    )(src, dst, ps[0], ps[1], ps[2], x,
      w.reshape(k3, cin * cout).astype(jnp.bfloat16), root,
      bias.reshape(1, cout))


def _pool_body(cl_ref, x_ref, out_ref, *, nb, cw, nc, ch):
    out_ref[...] = jnp.full((nc, ch), -jnp.inf, jnp.float32)
    iota_nc = lax.broadcasted_iota(jnp.int32, (nc, 1), 0)

    def body(i, carry):
        cl = cl_ref[pl.ds(i, 1), :]
        xc = x_ref[pl.ds(i * cw, cw), :]
        m = (iota_nc == cl)
        best = jnp.full((nc, ch), -jnp.inf, jnp.float32)
        for r in range(cw):
            cand = jnp.where(m[:, r:r + 1], xc[r:r + 1, :], -jnp.inf)
            best = jnp.maximum(best, cand)
        out_ref[...] = jnp.maximum(out_ref[...], best)
        return carry

    lax.fori_loop(0, nb, body, 0)
    o = out_ref[...]
    out_ref[...] = jnp.where(o > -jnp.inf, o, 0.0)


def _voxel_max_pool(x, cluster, nc, cw=8):
    nf, ch = x.shape
    nfp = _ceil_to(nf, cw)
    nb = nfp // cw
    cl = jnp.pad(cluster, (0, nfp - nf), constant_values=-1).reshape(nb, cw)
    xp = jnp.pad(x, ((0, nfp - nf), (0, 0)))
    body = functools.partial(_pool_body, nb=nb, cw=cw, nc=nc, ch=ch)
    return pl.pallas_call(
        body,
        out_shape=jax.ShapeDtypeStruct((nc, ch), jnp.float32),
    )(cl, xp)


def _unpool_body(clt_ref, xc_ref, xf_ref, wsk_ref, bsk_ref, out_ref, *,
                 nb, cw, nc, chc, chs):
    iota_nc = lax.broadcasted_iota(jnp.int32, (1, nc), 1)

    def body(i, carry):
        cl_cv = clt_ref[pl.ds(i, 1), :].reshape(cw, 1)
        g = (cl_cv == iota_nc).astype(jnp.float32)
        gath = jnp.dot(g, xc_ref[...], precision=_PREC)
        skip = (jnp.dot(xf_ref[pl.ds(i * cw, cw), :], wsk_ref[...],
                        precision=_PREC) + bsk_ref[...])
        out_ref[pl.ds(i * cw, cw), :chc] = gath
        out_ref[pl.ds(i * cw, cw), chc:chc + chs] = skip
        return carry

    lax.fori_loop(0, nb, body, 0)


def _unpool_concat(xc, cluster, xf, wsk, bsk, cw=500):
    nf, chf = xf.shape
    nc, chc = xc.shape
    chs = wsk.shape[1]
    nfp = _ceil_to(nf, cw)
    nb = nfp // cw
    cl = jnp.pad(cluster, (0, nfp - nf), constant_values=-1).reshape(nb, cw)
    xfp = jnp.pad(xf, ((0, nfp - nf), (0, 0)))
    body = functools.partial(_unpool_body, nb=nb, cw=cw, nc=nc, chc=chc,
                             chs=chs)
    out = pl.pallas_call(
        body,
        out_shape=jax.ShapeDtypeStruct((nfp, chc + chs), jnp.float32),
    )(cl, xc, xfp, wsk, bsk.reshape(1, chs))
    return out[:nf]


def _dense_elu_body(x_ref, w_ref, b_ref, out_ref):
    z = jnp.dot(x_ref[...], w_ref[...], precision=_PREC) + b_ref[...]
    out_ref[...] = jnp.where(z > 0.0, z, jnp.exp(jnp.minimum(z, 0.0)) - 1.0)


def _dense_elu(x, w, b):
    return pl.pallas_call(
        _dense_elu_body,
        out_shape=jax.ShapeDtypeStruct((x.shape[0], w.shape[1]),
                                       jnp.float32),
    )(x, w, b.reshape(1, -1))


def _logsoftmax_body(x_ref, w_ref, b_ref, out_ref):
    z = jnp.dot(x_ref[...], w_ref[...], precision=_PREC) + b_ref[...]
    m = jnp.max(z, axis=1, keepdims=True)
    s = z - m
    out_ref[...] = s - jnp.log(jnp.sum(jnp.exp(s), axis=1, keepdims=True))


def _dense_logsoftmax(x, w, b):
    return pl.pallas_call(
        _logsoftmax_body,
        out_shape=jax.ShapeDtypeStruct((x.shape[0], w.shape[1]),
                                       jnp.float32),
    )(x, w, b.reshape(1, -1))


def kernel(x, edge_index1, pseudo1, cluster1, edge_index2, pseudo2,
           cluster2, edge_index3, pseudo3, cluster3, edge_index4, pseudo4,
           W1, root1, b1, W2, root2, b2, W3, root3, b3, W42, root42, b42,
           Wfc1, bfc1, W5, root5, b5, W6, root6, b6, W7, root7, b7,
           Wsk1, bsk1, Wsk2, bsk2, Wsk3, bsk3, Wfc2, bfc2):
    x1 = _spline_conv(x, edge_index1, pseudo1, W1, root1, b1, True)
    x2 = _voxel_max_pool(x1, cluster1, 2500)
    x2 = _spline_conv(x2, edge_index2, pseudo2, W2, root2, b2, True)
    x3 = _voxel_max_pool(x2, cluster2, 600)
    x3 = _spline_conv(x3, edge_index3, pseudo3, W3, root3, b3, True)
    x4 = _voxel_max_pool(x3, cluster3, 150)
    x4 = _spline_conv(x4, edge_index4, pseudo4, W42, root42, b42, True)
    x4 = _dense_elu(x4, Wfc1, bfc1)
    x3c = _unpool_concat(x4, cluster3, x3, Wsk3, bsk3)
    x3c = _spline_conv(x3c, edge_index3, pseudo3, W5, root5, b5, True)
    x2c = _unpool_concat(x3c, cluster2, x2, Wsk2, bsk2)
    x2c = _spline_conv(x2c, edge_index2, pseudo2, W6, root6, b6, True)
    x1c = _unpool_concat(x2c, cluster1, x1, Wsk1, bsk1)
    x1c = _spline_conv(x1c, edge_index1, pseudo1, W7, root7, b7, True)
    return _dense_logsoftmax(x1c, Wfc2, bfc2)
